# baseline (speedup 1.0000x reference)
import jax
import jax.numpy as jnp
from jax import lax
from jax.experimental import pallas as pl
from jax.experimental.pallas import tpu as pltpu
from jax.experimental.pallas import tpu_sc as plsc

_MOVE_VOCAB_SIZE = 4672
_EMBED_DIM = 128
_BATCH = 4096
_HIST_LEN = 50
_B_BLK = 2

_NUM_SUBCORES = 16
_STAGE_MAIN = 288
_STAGE_REM_BASE = _STAGE_MAIN * _NUM_SUBCORES
_STAGE_REM = _MOVE_VOCAB_SIZE - _STAGE_REM_BASE


def _sc_gather(table, idx3):
    mesh = plsc.VectorSubcoreMesh(core_axis_name="core", subcore_axis_name="subcore")

    @pl.kernel(
        out_type=jax.ShapeDtypeStruct((_BATCH, _HIST_LEN, _EMBED_DIM), table.dtype),
        mesh=mesh,
        scratch_types=[
            pltpu.VMEM_SHARED((_MOVE_VOCAB_SIZE, _EMBED_DIM), jnp.float32),
        ],
    )
    def kern(tab_hbm, idx_hbm, out_hbm, tab_sp):
        sid = lax.axis_index("subcore")
        base = sid * _STAGE_MAIN
        pltpu.sync_copy(
            tab_hbm.at[pl.ds(base, _STAGE_MAIN)],
            tab_sp.at[pl.ds(base, _STAGE_MAIN)],
        )

        @pl.when(sid < _STAGE_REM // 8)
        def _():
            rbase = _STAGE_REM_BASE + sid * 8
            pltpu.sync_copy(
                tab_hbm.at[pl.ds(rbase, 8)],
                tab_sp.at[pl.ds(rbase, 8)],
            )

        plsc.subcore_barrier()

        def body(i_vmem, o_vmem):
            for b in range(_B_BLK):
                pltpu.sync_copy(tab_sp.at[i_vmem.at[b, 0]], o_vmem.at[b])

        pltpu.emit_pipeline(
            body,
            grid=(_BATCH // _B_BLK,),
            in_specs=[
                pl.BlockSpec((_B_BLK, 1, _HIST_LEN), index_map=lambda i: (i, 0, 0))
            ],
            out_specs=[
                pl.BlockSpec(
                    (_B_BLK, _HIST_LEN, _EMBED_DIM), index_map=lambda i: (i, 0, 0)
                )
            ],
            core_axis_name=("core", "subcore"),
            dimension_semantics=(pltpu.PARALLEL,),
        )(idx_hbm, out_hbm)

    return kern(table, idx3)


@jax.jit
def kernel(move_index, table):
    idx3 = move_index.reshape(_BATCH, 1, _HIST_LEN)
    return jax.lax.stop_gradient(_sc_gather(table, idx3))


# speedup vs baseline: 1.0143x; 1.0143x over previous
import jax
import jax.numpy as jnp
from jax import lax
from jax.experimental import pallas as pl
from jax.experimental.pallas import tpu as pltpu
from jax.experimental.pallas import tpu_sc as plsc

_MOVE_VOCAB_SIZE = 4672
_EMBED_DIM = 128
_BATCH = 4096
_HIST_LEN = 50
_B_BLK = 4

_NUM_SUBCORES = 16
_STAGE_MAIN = 288
_STAGE_REM_BASE = _STAGE_MAIN * _NUM_SUBCORES
_STAGE_REM = _MOVE_VOCAB_SIZE - _STAGE_REM_BASE


def _sc_gather(table, idx3):
    mesh = plsc.VectorSubcoreMesh(core_axis_name="core", subcore_axis_name="subcore")

    @pl.kernel(
        out_type=jax.ShapeDtypeStruct((_BATCH, _HIST_LEN, _EMBED_DIM), table.dtype),
        mesh=mesh,
        scratch_types=[
            pltpu.VMEM_SHARED((_MOVE_VOCAB_SIZE, _EMBED_DIM), jnp.float32),
        ],
    )
    def kern(tab_hbm, idx_hbm, out_hbm, tab_sp):
        sid = lax.axis_index("subcore")
        base = sid * _STAGE_MAIN
        pltpu.sync_copy(
            tab_hbm.at[pl.ds(base, _STAGE_MAIN)],
            tab_sp.at[pl.ds(base, _STAGE_MAIN)],
        )

        @pl.when(sid < _STAGE_REM // 8)
        def _():
            rbase = _STAGE_REM_BASE + sid * 8
            pltpu.sync_copy(
                tab_hbm.at[pl.ds(rbase, 8)],
                tab_sp.at[pl.ds(rbase, 8)],
            )

        plsc.subcore_barrier()

        def body(i_vmem, o_vmem):
            for b in range(_B_BLK):
                pltpu.sync_copy(tab_sp.at[i_vmem.at[b, 0]], o_vmem.at[b])

        pltpu.emit_pipeline(
            body,
            grid=(_BATCH // _B_BLK,),
            in_specs=[
                pl.BlockSpec((_B_BLK, 1, _HIST_LEN), index_map=lambda i: (i, 0, 0))
            ],
            out_specs=[
                pl.BlockSpec(
                    (_B_BLK, _HIST_LEN, _EMBED_DIM), index_map=lambda i: (i, 0, 0)
                )
            ],
            core_axis_name=("core", "subcore"),
            dimension_semantics=(pltpu.PARALLEL,),
        )(idx_hbm, out_hbm)

    return kern(table, idx3)


@jax.jit
def kernel(move_index, table):
    idx3 = move_index.reshape(_BATCH, 1, _HIST_LEN)
    return jax.lax.stop_gradient(_sc_gather(table, idx3))


# speedup vs baseline: 1.0619x; 1.0469x over previous
import jax
import jax.numpy as jnp
from jax import lax
from jax.experimental import pallas as pl
from jax.experimental.pallas import tpu as pltpu
from jax.experimental.pallas import tpu_sc as plsc

_MOVE_VOCAB_SIZE = 4672
_EMBED_DIM = 128
_BATCH = 4096
_HIST_LEN = 50
_B_BLK = 4

_NUM_SUBCORES = 16
_STAGE_MAIN = 288
_STAGE_REM_BASE = _STAGE_MAIN * _NUM_SUBCORES
_STAGE_REM = _MOVE_VOCAB_SIZE - _STAGE_REM_BASE


def _sc_gather(table, idx3):
    mesh = plsc.VectorSubcoreMesh(core_axis_name="core", subcore_axis_name="subcore")

    @pl.kernel(
        out_type=jax.ShapeDtypeStruct((_BATCH, _HIST_LEN, _EMBED_DIM), table.dtype),
        mesh=mesh,
        scratch_types=[
            pltpu.VMEM_SHARED((_MOVE_VOCAB_SIZE, _EMBED_DIM), jnp.float32),
            pltpu.SemaphoreType.DMA,
        ],
    )
    def kern(tab_hbm, idx_hbm, out_hbm, tab_sp, gsem):
        sid = lax.axis_index("subcore")
        base = sid * _STAGE_MAIN
        pltpu.sync_copy(
            tab_hbm.at[pl.ds(base, _STAGE_MAIN)],
            tab_sp.at[pl.ds(base, _STAGE_MAIN)],
        )

        @pl.when(sid < _STAGE_REM // 8)
        def _():
            rbase = _STAGE_REM_BASE + sid * 8
            pltpu.sync_copy(
                tab_hbm.at[pl.ds(rbase, 8)],
                tab_sp.at[pl.ds(rbase, 8)],
            )

        plsc.subcore_barrier()

        def body(i_vmem, o_vmem):
            handles = [
                pltpu.async_copy(tab_sp.at[i_vmem.at[b, 0]], o_vmem.at[b], gsem)
                for b in range(_B_BLK)
            ]
            for h in handles:
                h.wait()

        pltpu.emit_pipeline(
            body,
            grid=(_BATCH // _B_BLK,),
            in_specs=[
                pl.BlockSpec((_B_BLK, 1, _HIST_LEN), index_map=lambda i: (i, 0, 0))
            ],
            out_specs=[
                pl.BlockSpec(
                    (_B_BLK, _HIST_LEN, _EMBED_DIM), index_map=lambda i: (i, 0, 0)
                )
            ],
            core_axis_name=("core", "subcore"),
            dimension_semantics=(pltpu.PARALLEL,),
        )(idx_hbm, out_hbm)

    return kern(table, idx3)


@jax.jit
def kernel(move_index, table):
    idx3 = move_index.reshape(_BATCH, 1, _HIST_LEN)
    return jax.lax.stop_gradient(_sc_gather(table, idx3))
